# baseline (speedup 1.0000x reference)
import functools

import jax
import jax.numpy as jnp
from jax import lax
from jax.experimental import pallas as pl
from jax.experimental.pallas import tpu as pltpu
from jax.experimental.pallas import tpu_sc as plsc

N = 640 * 480
D = 64
L = 16
NC = 2
NS = 16
NW = NC * NS
CH = 128
MPW = 384
NPW = 3200
NB_MATCH = 10000
NB_NONMATCH = 100000
MARGIN = 0.5


def _sc_body(tabA, tabB, mA, mB, nA, nB, out,
             idx_ma, idx_mb, idx_na, idx_nb,
             bufA, bufB, vec_v, red_v, shared, semA, semB):
    c = lax.axis_index("c")
    s = lax.axis_index("s")
    w = c * NS + s

    pltpu.sync_copy(mA.at[w], idx_ma)
    pltpu.sync_copy(mB.at[w], idx_mb)
    pltpu.sync_copy(nA.at[w], idx_na)
    pltpu.sync_copy(nB.at[w], idx_nb)

    vm = NB_MATCH - w * MPW
    vn = NB_NONMATCH - w * NPW

    zeros = jnp.zeros((L,), jnp.float32)
    ones = jnp.ones((L,), jnp.float32)
    margin_v = jnp.full((L,), MARGIN, jnp.float32)

    def neutralize(valid, k, b_fill):
        lo = jnp.clip(valid - k * CH, 0, CH)

        def zrow(r, carry):
            for j in range(D // L):
                bufA[r, pl.ds(j * L, L)] = zeros
                bufB[r, pl.ds(j * L, L)] = b_fill
            return carry

        lax.fori_loop(lo, CH, zrow, 0)

    acc_m = zeros
    for k in range(MPW // CH):
        ca = pltpu.async_copy(tabA.at[idx_ma.at[pl.ds(k * CH, CH)]], bufA, semA)
        cb = pltpu.async_copy(tabB.at[idx_mb.at[pl.ds(k * CH, CH)]], bufB, semB)
        ca.wait()
        cb.wait()
        neutralize(vm, k, zeros)

        def mrow(r, acc):
            t = None
            for j in range(D // L):
                a = bufA[r, pl.ds(j * L, L)]
                b = bufB[r, pl.ds(j * L, L)]
                d = a - b
                t = d * d if t is None else t + d * d
            return acc + t

        acc_m = lax.fori_loop(0, CH, mrow, acc_m)

    acc_s = zeros
    acc_c = zeros
    for k in range(NPW // CH):
        ca = pltpu.async_copy(tabA.at[idx_na.at[pl.ds(k * CH, CH)]], bufA, semA)
        cb = pltpu.async_copy(tabB.at[idx_nb.at[pl.ds(k * CH, CH)]], bufB, semB)
        ca.wait()
        cb.wait()
        neutralize(vn, k, ones)

        def nrow(r, carry):
            ss, cc = carry
            ts = None
            tc = None
            for j in range(D // L):
                a = bufA[r, pl.ds(j * L, L)]
                b = bufB[r, pl.ds(j * L, L)]
                d = a - b
                d2 = d * d
                h = jnp.maximum(margin_v - d2, zeros)
                ind = jnp.where(d2 < margin_v, ones, zeros)
                ts = h if ts is None else ts + h
                tc = ind if tc is None else tc + ind
            return ss + ts, cc + tc

        acc_s, acc_c = lax.fori_loop(0, CH, nrow, (acc_s, acc_c))

    ms = jnp.sum(acc_m)
    ss = jnp.sum(acc_s)
    cs = jnp.sum(acc_c)
    lane = lax.iota(jnp.int32, L)
    vec = (jnp.where(lane == 0, jnp.full((L,), ms), zeros)
           + jnp.where(lane == 1, jnp.full((L,), ss), zeros)
           + jnp.where(lane == 2, jnp.full((L,), cs), zeros))
    vec_v[...] = vec

    pltpu.sync_copy(vec_v, shared.at[s])
    plsc.subcore_barrier()

    @pl.when(s == 0)
    def _():
        pltpu.sync_copy(shared, red_v)
        tot = red_v[0]
        for i in range(1, NS):
            tot = tot + red_v[i]
        vec_v[...] = tot
        pltpu.sync_copy(vec_v, out.at[c])


_sc_loss = functools.partial(
    pl.kernel,
    out_type=jax.ShapeDtypeStruct((NC, L), jnp.float32),
    mesh=plsc.VectorSubcoreMesh(core_axis_name="c", subcore_axis_name="s"),
    compiler_params=pltpu.CompilerParams(use_tc_tiling_on_sc=False, needs_layout_passes=False),
    scratch_types=[
        pltpu.VMEM((MPW,), jnp.int32),
        pltpu.VMEM((MPW,), jnp.int32),
        pltpu.VMEM((NPW,), jnp.int32),
        pltpu.VMEM((NPW,), jnp.int32),
        pltpu.VMEM((CH, D), jnp.float32),
        pltpu.VMEM((CH, D), jnp.float32),
        pltpu.VMEM((L,), jnp.float32),
        pltpu.VMEM((NS, L), jnp.float32),
        pltpu.VMEM_SHARED((NS, L), jnp.float32),
        pltpu.SemaphoreType.DMA,
        pltpu.SemaphoreType.DMA,
    ],
)(_sc_body)


def _pad_split(idx, per_w):
    idx = idx.astype(jnp.int32)
    return jnp.concatenate(
        [idx, jnp.zeros((NW * per_w - idx.shape[0],), jnp.int32)]
    ).reshape(NW, per_w)


@jax.jit
def kernel(out_A, out_B, match_A, match_B, non_match_A, non_match_B):
    tabA = out_A.reshape(N, D)
    tabB = out_B.reshape(N, D)
    mA = _pad_split(match_A, MPW)
    mB = _pad_split(match_B, MPW)
    nA = _pad_split(non_match_A, NPW)
    nB = _pad_split(non_match_B, NPW)

    part = _sc_loss(tabA, tabB, mA, mB, nA, nB)
    p = part[0] + part[1]
    match_loss = p[0] * jnp.float32(1.0 / NB_MATCH)
    non_match_loss = p[1] / p[2]
    return match_loss + non_match_loss, match_loss, non_match_loss


# speedup vs baseline: 1.0522x; 1.0522x over previous
import functools

import jax
import jax.numpy as jnp
from jax import lax
from jax.experimental import pallas as pl
from jax.experimental.pallas import tpu as pltpu
from jax.experimental.pallas import tpu_sc as plsc

N = 640 * 480
D = 64
L = 16
NC = 2
NS = 16
NW = NC * NS
CH = 128
U = 4
MPW = 384
NPW = 3200
MK = MPW // CH
NK = NPW // CH
NB_MATCH = 10000
NB_NONMATCH = 100000
MARGIN = 0.5


def _sc_body(tabA, tabB, mA, mB, nA, nB, out,
             idx_ma, idx_mb, idx_na, idx_nb,
             bufsA, bufsB, vec_v, red_v, shared, semsA, semsB):
    c = lax.axis_index("c")
    s = lax.axis_index("s")
    w = c * NS + s

    pltpu.sync_copy(mA.at[w], idx_ma)
    pltpu.sync_copy(mB.at[w], idx_mb)
    pltpu.sync_copy(nA.at[w], idx_na)
    pltpu.sync_copy(nB.at[w], idx_nb)

    vm = NB_MATCH - w * MPW
    vn = NB_NONMATCH - w * NPW

    zeros = jnp.zeros((L,), jnp.float32)
    ones = jnp.ones((L,), jnp.float32)
    margin_v = jnp.full((L,), MARGIN, jnp.float32)

    chunks = [("m", k) for k in range(MK)] + [("n", k) for k in range(NK)]

    def start(i):
        phase, k = chunks[i]
        p = i % 2
        ia, ib = (idx_ma, idx_mb) if phase == "m" else (idx_na, idx_nb)
        ca = pltpu.async_copy(tabA.at[ia.at[pl.ds(k * CH, CH)]], bufsA.at[p], semsA[p])
        cb = pltpu.async_copy(tabB.at[ib.at[pl.ds(k * CH, CH)]], bufsB.at[p], semsB[p])
        return ca, cb

    def neutralize(valid, k, p, b_fill):
        lo = jnp.clip(valid - k * CH, 0, CH)

        def zrow(r, carry):
            for j in range(D // L):
                bufsA[p, r, pl.ds(j * L, L)] = zeros
                bufsB[p, r, pl.ds(j * L, L)] = b_fill
            return carry

        lax.fori_loop(lo, CH, zrow, 0)

    acc_m = zeros
    acc_s = zeros
    acc_c = zeros

    pend = start(0)
    for i in range(len(chunks)):
        phase, k = chunks[i]
        p = i % 2
        nxt = start(i + 1) if i + 1 < len(chunks) else None
        pend[0].wait()
        pend[1].wait()
        pend = nxt

        if phase == "m":
            neutralize(vm, k, p, zeros)

            def mrow(t, acc, p=p):
                tacc = None
                for u in range(U):
                    r = t * U + u
                    for j in range(D // L):
                        a = bufsA[p, r, pl.ds(j * L, L)]
                        b = bufsB[p, r, pl.ds(j * L, L)]
                        d = a - b
                        tacc = d * d if tacc is None else tacc + d * d
                return acc + tacc

            acc_m = lax.fori_loop(0, CH // U, mrow, acc_m)
        else:
            neutralize(vn, k, p, ones)

            def nrow(t, carry, p=p):
                ss, cc = carry
                ts = None
                tc = None
                for u in range(U):
                    r = t * U + u
                    for j in range(D // L):
                        a = bufsA[p, r, pl.ds(j * L, L)]
                        b = bufsB[p, r, pl.ds(j * L, L)]
                        d = a - b
                        d2 = d * d
                        h = jnp.maximum(margin_v - d2, zeros)
                        ind = jnp.where(d2 < margin_v, ones, zeros)
                        ts = h if ts is None else ts + h
                        tc = ind if tc is None else tc + ind
                return ss + ts, cc + tc

            acc_s, acc_c = lax.fori_loop(0, CH // U, nrow, (acc_s, acc_c))

    ms = jnp.sum(acc_m)
    ss = jnp.sum(acc_s)
    cs = jnp.sum(acc_c)
    lane = lax.iota(jnp.int32, L)
    vec = (jnp.where(lane == 0, jnp.full((L,), ms), zeros)
           + jnp.where(lane == 1, jnp.full((L,), ss), zeros)
           + jnp.where(lane == 2, jnp.full((L,), cs), zeros))
    vec_v[...] = vec

    pltpu.sync_copy(vec_v, shared.at[s])
    plsc.subcore_barrier()

    @pl.when(s == 0)
    def _():
        pltpu.sync_copy(shared, red_v)
        tot = red_v[0]
        for i in range(1, NS):
            tot = tot + red_v[i]
        vec_v[...] = tot
        pltpu.sync_copy(vec_v, out.at[c])


_sc_loss = functools.partial(
    pl.kernel,
    out_type=jax.ShapeDtypeStruct((NC, L), jnp.float32),
    mesh=plsc.VectorSubcoreMesh(core_axis_name="c", subcore_axis_name="s"),
    compiler_params=pltpu.CompilerParams(use_tc_tiling_on_sc=False, needs_layout_passes=False),
    scratch_types=[
        pltpu.VMEM((MPW,), jnp.int32),
        pltpu.VMEM((MPW,), jnp.int32),
        pltpu.VMEM((NPW,), jnp.int32),
        pltpu.VMEM((NPW,), jnp.int32),
        pltpu.VMEM((2, CH, D), jnp.float32),
        pltpu.VMEM((2, CH, D), jnp.float32),
        pltpu.VMEM((L,), jnp.float32),
        pltpu.VMEM((NS, L), jnp.float32),
        pltpu.VMEM_SHARED((NS, L), jnp.float32),
        [pltpu.SemaphoreType.DMA, pltpu.SemaphoreType.DMA],
        [pltpu.SemaphoreType.DMA, pltpu.SemaphoreType.DMA],
    ],
)(_sc_body)


def _pad_split(idx, per_w):
    idx = idx.astype(jnp.int32)
    return jnp.concatenate(
        [idx, jnp.zeros((NW * per_w - idx.shape[0],), jnp.int32)]
    ).reshape(NW, per_w)


@jax.jit
def kernel(out_A, out_B, match_A, match_B, non_match_A, non_match_B):
    tabA = out_A.reshape(N, D)
    tabB = out_B.reshape(N, D)
    mA = _pad_split(match_A, MPW)
    mB = _pad_split(match_B, MPW)
    nA = _pad_split(non_match_A, NPW)
    nB = _pad_split(non_match_B, NPW)

    part = _sc_loss(tabA, tabB, mA, mB, nA, nB)
    p = part[0] + part[1]
    match_loss = p[0] * jnp.float32(1.0 / NB_MATCH)
    non_match_loss = p[1] / p[2]
    return match_loss + non_match_loss, match_loss, non_match_loss


# speedup vs baseline: 1.1579x; 1.1004x over previous
import functools

import jax
import jax.numpy as jnp
from jax import lax
from jax.experimental import pallas as pl
from jax.experimental.pallas import tpu as pltpu
from jax.experimental.pallas import tpu_sc as plsc

N = 640 * 480
D = 64
L = 16
NC = 2
NS = 16
NW = NC * NS
CH = 320
U = 4
NBUF = 2
MPW = 320
NPW = 3200
MK = MPW // CH
NK = NPW // CH
NB_MATCH = 10000
NB_NONMATCH = 100000
MARGIN = 0.5


def _sc_body(tabA, tabB, mA, mB, nA, nB, out,
             idx_ma, idx_mb, idx_na, idx_nb,
             bufsA, bufsB, vec_v, red_v, shared, semsA, semsB):
    c = lax.axis_index("c")
    s = lax.axis_index("s")
    w = c * NS + s

    pltpu.sync_copy(mA.at[pl.ds(w * MPW, MPW)], idx_ma)
    pltpu.sync_copy(mB.at[pl.ds(w * MPW, MPW)], idx_mb)
    pltpu.sync_copy(nA.at[pl.ds(w * NPW, NPW)], idx_na)
    pltpu.sync_copy(nB.at[pl.ds(w * NPW, NPW)], idx_nb)

    vm = NB_MATCH - w * MPW
    vn = NB_NONMATCH - w * NPW

    zeros = jnp.zeros((L,), jnp.float32)
    ones = jnp.ones((L,), jnp.float32)
    margin_v = jnp.full((L,), MARGIN, jnp.float32)

    chunks = [("m", k) for k in range(MK)] + [("n", k) for k in range(NK)]

    def start(i):
        phase, k = chunks[i]
        p = i % NBUF
        ia, ib = (idx_ma, idx_mb) if phase == "m" else (idx_na, idx_nb)
        copies = []
        for t in range(CH // L):
            iva = ia[pl.ds(k * CH + t * L, L)]
            ivb = ib[pl.ds(k * CH + t * L, L)]
            copies.append(pltpu.async_copy(
                tabA.at[iva], bufsA.at[p, pl.ds(t * L, L)], semsA[p]))
            copies.append(pltpu.async_copy(
                tabB.at[ivb], bufsB.at[p, pl.ds(t * L, L)], semsB[p]))
        return copies

    def neutralize(valid, k, p, b_fill):
        lo = jnp.clip(valid - k * CH, 0, CH)

        def zrow(r, carry):
            for j in range(D // L):
                bufsA[p, r, pl.ds(j * L, L)] = zeros
                bufsB[p, r, pl.ds(j * L, L)] = b_fill
            return carry

        lax.fori_loop(lo, CH, zrow, 0)

    acc_m = zeros
    acc_s = zeros
    acc_c = zeros

    nch = len(chunks)
    pend = [start(i) for i in range(NBUF - 1)]
    for i in range(nch):
        phase, k = chunks[i]
        p = i % NBUF
        if i + NBUF - 1 < nch:
            pend.append(start(i + NBUF - 1))
        for cp in pend.pop(0):
            cp.wait()

        if phase == "m":
            neutralize(vm, k, p, zeros)

            def mrow(t, acc, p=p):
                tacc = None
                for u in range(U):
                    r = t * U + u
                    for j in range(D // L):
                        a = bufsA[p, r, pl.ds(j * L, L)]
                        b = bufsB[p, r, pl.ds(j * L, L)]
                        d = a - b
                        tacc = d * d if tacc is None else tacc + d * d
                return acc + tacc

            acc_m = lax.fori_loop(0, CH // U, mrow, acc_m)
        else:
            neutralize(vn, k, p, ones)

            def nrow(t, carry, p=p):
                ss, cc = carry
                ts = None
                tc = None
                for u in range(U):
                    r = t * U + u
                    for j in range(D // L):
                        a = bufsA[p, r, pl.ds(j * L, L)]
                        b = bufsB[p, r, pl.ds(j * L, L)]
                        d = a - b
                        d2 = d * d
                        h = jnp.maximum(margin_v - d2, zeros)
                        ind = jnp.where(d2 < margin_v, ones, zeros)
                        ts = h if ts is None else ts + h
                        tc = ind if tc is None else tc + ind
                return ss + ts, cc + tc

            acc_s, acc_c = lax.fori_loop(0, CH // U, nrow, (acc_s, acc_c))

    ms = jnp.sum(acc_m)
    ss = jnp.sum(acc_s)
    cs = jnp.sum(acc_c)
    lane = lax.iota(jnp.int32, L)
    vec = (jnp.where(lane == 0, jnp.full((L,), ms), zeros)
           + jnp.where(lane == 1, jnp.full((L,), ss), zeros)
           + jnp.where(lane == 2, jnp.full((L,), cs), zeros))
    vec_v[...] = vec

    pltpu.sync_copy(vec_v, shared.at[s])
    plsc.subcore_barrier()

    @pl.when(s == 0)
    def _():
        pltpu.sync_copy(shared, red_v)
        tot = red_v[0]
        for i in range(1, NS):
            tot = tot + red_v[i]
        vec_v[...] = tot
        pltpu.sync_copy(vec_v, out.at[c])


_sc_loss = functools.partial(
    pl.kernel,
    out_type=jax.ShapeDtypeStruct((NC, L), jnp.float32),
    mesh=plsc.VectorSubcoreMesh(core_axis_name="c", subcore_axis_name="s"),
    compiler_params=pltpu.CompilerParams(use_tc_tiling_on_sc=False, needs_layout_passes=False),
    scratch_types=[
        pltpu.VMEM((MPW,), jnp.int32),
        pltpu.VMEM((MPW,), jnp.int32),
        pltpu.VMEM((NPW,), jnp.int32),
        pltpu.VMEM((NPW,), jnp.int32),
        pltpu.VMEM((NBUF, CH, D), jnp.float32),
        pltpu.VMEM((NBUF, CH, D), jnp.float32),
        pltpu.VMEM((L,), jnp.float32),
        pltpu.VMEM((NS, L), jnp.float32),
        pltpu.VMEM_SHARED((NS, L), jnp.float32),
        [pltpu.SemaphoreType.DMA] * NBUF,
        [pltpu.SemaphoreType.DMA] * NBUF,
    ],
)(_sc_body)


def _pad_split(idx, per_w):
    idx = idx.astype(jnp.int32)
    return jnp.concatenate(
        [idx, jnp.zeros((NW * per_w - idx.shape[0],), jnp.int32)])


@jax.jit
def kernel(out_A, out_B, match_A, match_B, non_match_A, non_match_B):
    tabA = out_A.reshape(N, D)
    tabB = out_B.reshape(N, D)
    mA = _pad_split(match_A, MPW)
    mB = _pad_split(match_B, MPW)
    nA = _pad_split(non_match_A, NPW)
    nB = _pad_split(non_match_B, NPW)

    part = _sc_loss(tabA, tabB, mA, mB, nA, nB)
    p = part[0] + part[1]
    match_loss = p[0] * jnp.float32(1.0 / NB_MATCH)
    non_match_loss = p[1] / p[2]
    return match_loss + non_match_loss, match_loss, non_match_loss
